# baseline (speedup 1.0000x reference)
import functools

import jax
import jax.numpy as jnp
from jax import lax
from jax.experimental import pallas as pl
from jax.experimental.pallas import tpu as pltpu
from jax.experimental.pallas import tpu_sc as plsc

_B = 4096 * 200
_D = 64
_NW = 32
_BPW = _B // _NW
_CHUNK = 512
_NCH = _BPW // _CHUNK

_mesh = plsc.VectorSubcoreMesh(core_axis_name="c", subcore_axis_name="s")


@functools.partial(
    pl.kernel,
    out_type=jax.ShapeDtypeStruct((_B, _D), jnp.float32),
    mesh=_mesh,
    scratch_types=[
        pltpu.VMEM((_CHUNK,), jnp.int32),
        pltpu.VMEM((_CHUNK, _D), jnp.float32),
        pltpu.SemaphoreType.DMA,
    ],
    compiler_params=pltpu.CompilerParams(use_tc_tiling_on_sc=False),
)
def _gather_kernel(ids_hbm, table_hbm, out_hbm, idx_v, rows_v, sem):
    wid = lax.axis_index("s") * 2 + lax.axis_index("c")
    base = wid * _BPW

    def body(g, carry):
        off = base + g * _CHUNK
        pltpu.sync_copy(ids_hbm.at[pl.ds(off, _CHUNK)], idx_v)
        pltpu.async_copy(table_hbm.at[idx_v], rows_v, sem).wait()
        pltpu.sync_copy(rows_v, out_hbm.at[pl.ds(off, _CHUNK)])
        return carry

    lax.fori_loop(0, _NCH, body, 0)


def kernel(item_ids, table):
    num_embeddings = table.shape[0]
    ids = jnp.clip(item_ids.reshape(-1), 0, num_embeddings - 1)
    out = _gather_kernel(ids, table)
    return out.reshape(item_ids.shape + (table.shape[1],))


# speedup vs baseline: 1.0394x; 1.0394x over previous
import functools

import jax
import jax.numpy as jnp
from jax import lax
from jax.experimental import pallas as pl
from jax.experimental.pallas import tpu as pltpu
from jax.experimental.pallas import tpu_sc as plsc

_B = 4096 * 200
_D = 64
_NW = 32
_BPW = _B // _NW
_CHUNK = 512
_NCH = _BPW // _CHUNK
_NT = _NCH // 2

_mesh = plsc.VectorSubcoreMesh(core_axis_name="c", subcore_axis_name="s")


@functools.partial(
    pl.kernel,
    out_type=jax.ShapeDtypeStruct((_B, _D), jnp.float32),
    mesh=_mesh,
    scratch_types=[
        pltpu.VMEM((2, _CHUNK), jnp.int32),
        pltpu.VMEM((_CHUNK, _D), jnp.float32),
        pltpu.VMEM((_CHUNK, _D), jnp.float32),
        pltpu.SemaphoreType.DMA((2,)),
        pltpu.SemaphoreType.DMA((2,)),
        pltpu.SemaphoreType.DMA((2,)),
    ],
    compiler_params=pltpu.CompilerParams(use_tc_tiling_on_sc=False),
)
def _gather_kernel(ids_hbm, table_hbm, out_hbm, idx_v, rows0, rows1,
                   isem, gsem, osem):
    wid = lax.axis_index("s") * 2 + lax.axis_index("c")
    base = wid * _BPW
    rows = (rows0, rows1)

    def fire_idx(g, b):
        pltpu.async_copy(ids_hbm.at[pl.ds(base + g * _CHUNK, _CHUNK)],
                         idx_v.at[b], isem.at[b])

    def fire_gather(b):
        pltpu.async_copy(table_hbm.at[idx_v.at[b]], rows[b], gsem.at[b])

    def wait_gather(b):
        pltpu.make_async_copy(table_hbm.at[idx_v.at[b]], rows[b],
                              gsem.at[b]).wait()

    def wait_idx(b):
        pltpu.make_async_copy(ids_hbm.at[pl.ds(0, _CHUNK)], idx_v.at[b],
                              isem.at[b]).wait()

    def fire_out(g, b):
        pltpu.async_copy(rows[b], out_hbm.at[pl.ds(base + g * _CHUNK, _CHUNK)],
                         osem.at[b])

    def wait_out(b):
        pltpu.make_async_copy(rows[b], out_hbm.at[pl.ds(0, _CHUNK)],
                              osem.at[b]).wait()

    fire_idx(0, 0)
    fire_idx(1, 1)
    for b in (0, 1):
        wait_idx(b)
        fire_gather(b)
        wait_gather(b)
        fire_out(b, b)
        fire_idx(b + 2, b)

    def body(t, carry):
        for b in (0, 1):
            g = 2 * t + b
            wait_idx(b)
            wait_out(b)
            fire_gather(b)
            wait_gather(b)
            fire_out(g, b)

            @pl.when(g + 2 < _NCH)
            def _():
                fire_idx(g + 2, b)
        return carry

    lax.fori_loop(1, _NT, body, 0)
    wait_out(0)
    wait_out(1)


def kernel(item_ids, table):
    num_embeddings = table.shape[0]
    ids = jnp.clip(item_ids.reshape(-1), 0, num_embeddings - 1)
    out = _gather_kernel(ids, table)
    return out.reshape(item_ids.shape + (table.shape[1],))


# speedup vs baseline: 1.0452x; 1.0056x over previous
import functools

import jax
import jax.numpy as jnp
from jax import lax
from jax.experimental import pallas as pl
from jax.experimental.pallas import tpu as pltpu
from jax.experimental.pallas import tpu_sc as plsc

_B = 4096 * 200
_D = 64
_NW = 32
_BPW = _B // _NW
_C = 128
_NBUF = 8
_LAG = 6
_NCH = _BPW // _C
_NT = _NCH // _NBUF

_mesh = plsc.VectorSubcoreMesh(core_axis_name="c", subcore_axis_name="s")


@functools.partial(
    pl.kernel,
    out_type=jax.ShapeDtypeStruct((_B, _D), jnp.float32),
    mesh=_mesh,
    scratch_types=[
        pltpu.VMEM((_NBUF, _C), jnp.int32),
        pltpu.VMEM((_NBUF, _C, _D), jnp.float32),
        pltpu.SemaphoreType.DMA((_NBUF,)),
        pltpu.SemaphoreType.DMA((_NBUF,)),
        pltpu.SemaphoreType.DMA((_NBUF,)),
    ],
    compiler_params=pltpu.CompilerParams(use_tc_tiling_on_sc=False),
)
def _gather_kernel(ids_hbm, table_hbm, out_hbm, idx_v, rows_v,
                   isem, gsem, osem):
    wid = lax.axis_index("s") * 2 + lax.axis_index("c")
    base = wid * _BPW

    def fire_idx(g, b):
        pltpu.async_copy(ids_hbm.at[pl.ds(base + g * _C, _C)],
                         idx_v.at[b], isem.at[b])

    def wait_idx(b):
        pltpu.make_async_copy(ids_hbm.at[pl.ds(0, _C)], idx_v.at[b],
                              isem.at[b]).wait()

    def fire_gather(b):
        pltpu.async_copy(table_hbm.at[idx_v.at[b]], rows_v.at[b], gsem.at[b])

    def wait_gather(b):
        pltpu.make_async_copy(table_hbm.at[idx_v.at[b]], rows_v.at[b],
                              gsem.at[b]).wait()

    def fire_out(g, b):
        pltpu.async_copy(rows_v.at[b], out_hbm.at[pl.ds(base + g * _C, _C)],
                         osem.at[b])

    def wait_out(b):
        pltpu.make_async_copy(rows_v.at[b], out_hbm.at[pl.ds(0, _C)],
                              osem.at[b]).wait()

    def drain(g, b):
        wait_gather(b)
        fire_out(g, b)
        if isinstance(g, int):
            if g + _NBUF < _NCH:
                fire_idx(g + _NBUF, b)
        else:
            @pl.when(g + _NBUF < _NCH)
            def _():
                fire_idx(g + _NBUF, b)

    for b in range(_NBUF):
        fire_idx(b, b)
    for j in range(_NBUF):
        wait_idx(j)
        fire_gather(j)
        if j >= _LAG:
            drain(j - _LAG, j - _LAG)

    def body(t, carry):
        for b in range(_NBUF):
            g = t * _NBUF + b
            wait_out(b)
            wait_idx(b)
            fire_gather(b)
            drain(g - _LAG, (b - _LAG) % _NBUF)
        return carry

    lax.fori_loop(1, _NT, body, 0)

    for j in range(_NCH - _LAG, _NCH):
        drain(j, j % _NBUF)
    for b in range(_NBUF):
        wait_out(b)


def kernel(item_ids, table):
    num_embeddings = table.shape[0]
    ids = jnp.clip(item_ids.reshape(-1), 0, num_embeddings - 1)
    out = _gather_kernel(ids, table)
    return out.reshape(item_ids.shape + (table.shape[1],))
